# Initial kernel scaffold; baseline (speedup 1.0000x reference)
import jax
import jax.numpy as jnp
from jax.experimental import pallas as pl


def kernel(x, params, edge_index):
    raise NotImplementedError("write your pallas kernel here")


# speedup vs baseline: 29.1611x; 29.1611x over previous
import jax
import jax.numpy as jnp
from jax.experimental import pallas as pl

H = 8
C = 40
SKC = 320
ENDC = 640
ODIM = 12
DIL = [1, 2, 1, 2, 1, 2, 1, 2]
NP = 208
INVBN = 1.0 / (1.0 + 1e-5) ** 0.5
F32 = jnp.float32


def _stem_k(x0_ref, x1_ref, ss_ref, sc_ref, bs_ref, bc_ref, out_ref):
    a = jnp.dot(x0_ref[...], ss_ref[...], preferred_element_type=F32) + bs_ref[...]
    b = jnp.dot(x1_ref[...], sc_ref[...], preferred_element_type=F32) + bc_ref[...]
    out_ref[...] = a + jnp.where(b >= 0, b, 0.01 * b)


def _tcn_k(h_ref, wf_ref, wg_ref, bf_ref, bg_ref, out_ref):
    hv = h_ref[...]
    f = jnp.tanh(jnp.dot(hv, wf_ref[...], preferred_element_type=F32) + bf_ref[...])
    g = jax.nn.sigmoid(jnp.dot(hv, wg_ref[...], preferred_element_type=F32) + bg_ref[...])
    out_ref[...] = f * g


def _mask_k(src_ref, dst_ref, out_ref):
    s = src_ref[...]
    d = dst_ref[...]
    iota = jax.lax.broadcasted_iota(jnp.int32, (s.shape[0], NP), 1)
    sh = (iota == s).astype(F32)
    dh = (iota == d).astype(F32)
    out_ref[...] = jax.lax.dot_general(
        dh, sh, (((0,), (0,)), ((), ())), preferred_element_type=F32)


def _gat_heads(hg, w_ref, al_ref, ar_ref, cnt, valid, dout):
    acc = jnp.zeros((NP, dout), F32)
    for h in range(H):
        w = w_ref[h]
        feat = jnp.dot(hg, w, preferred_element_type=F32)
        al = al_ref[h:h + 1, :]
        ar = ar_ref[h:h + 1, :]
        er = jax.lax.dot_general(feat, ar, (((1,), (1,)), ((), ())),
                                 preferred_element_type=F32)
        el = jax.lax.dot_general(al, feat, (((1,), (1,)), ((), ())),
                                 preferred_element_type=F32)
        e = er + el
        e = jnp.where(e >= 0, e, 0.2 * e)
        m = jnp.max(jnp.where(valid, e, -1e30), axis=1, keepdims=True)
        sx = cnt * jnp.exp(jnp.where(valid, e - m, -30.0))
        ss = jnp.sum(sx, axis=1, keepdims=True)
        alpha = sx / jnp.where(ss > 0, ss, 1.0)
        rst = jnp.dot(alpha, feat, preferred_element_type=F32)
        acc = acc + jnp.where(rst > 0, rst, jnp.exp(jnp.minimum(rst, 0.0)) - 1.0)
    return acc * (1.0 / H)


def _gata_k(hg_ref, w_ref, al_ref, ar_ref, cnt_ref, out_ref):
    cnt = cnt_ref[...]
    out_ref[...] = _gat_heads(hg_ref[...], w_ref, al_ref, ar_ref,
                              cnt, cnt > 0, out_ref.shape[1])


def _gatb_k(hg_ref, w_ref, al_ref, ar_ref, cnt_ref, hprev_ref, res_ref, out_ref):
    cnt = cnt_ref[...]
    acc = _gat_heads(hg_ref[...], w_ref, al_ref, ar_ref,
                     cnt, cnt > 0, out_ref.shape[1])
    out_ref[...] = (acc + hprev_ref[...] + res_ref[...]) * INVBN


def _head_k(hl_ref, wsk_ref, bsk_ref, w1_ref, b1_ref, w2_ref, b2_ref, out_ref):
    skip = jnp.dot(hl_ref[...], wsk_ref[...], preferred_element_type=F32) + bsk_ref[...]
    o = jnp.maximum(skip, 0.0)
    o = jnp.maximum(jnp.dot(o, w1_ref[...], preferred_element_type=F32) + b1_ref[...], 0.0)
    out_ref[...] = jnp.dot(o, w2_ref[...], preferred_element_type=F32) + b2_ref[...]


def _rows_spec(cols):
    return pl.BlockSpec((NP, cols), lambda r: (r, 0))


def _const_spec(shape):
    nd = len(shape)
    return pl.BlockSpec(shape, lambda r: (0,) * nd)


def _stem(x0, x1, ss, sc, bs, bc, bn):
    t = x0.shape[1]
    ct = ss.shape[1]
    return pl.pallas_call(
        _stem_k,
        grid=(bn // NP,),
        in_specs=[_rows_spec(t), _rows_spec(t), _const_spec(ss.shape),
                  _const_spec(sc.shape), _const_spec(bs.shape), _const_spec(bc.shape)],
        out_specs=_rows_spec(ct),
        out_shape=jax.ShapeDtypeStruct((bn, ct), F32),
    )(x0, x1, ss, sc, bs, bc)


def _tcn(h, wf, wg, bf, bg):
    bn, ctin = h.shape
    ctout = wf.shape[1]
    return pl.pallas_call(
        _tcn_k,
        grid=(bn // NP,),
        in_specs=[_rows_spec(ctin), _const_spec(wf.shape), _const_spec(wg.shape),
                  _const_spec(bf.shape), _const_spec(bg.shape)],
        out_specs=_rows_spec(ctout),
        out_shape=jax.ShapeDtypeStruct((bn, ctout), F32),
    )(h, wf, wg, bf, bg)


def _mask(srcp, dstp):
    ep = srcp.shape[0]
    return pl.pallas_call(
        _mask_k,
        in_specs=[pl.BlockSpec((ep, 1), lambda: (0, 0)),
                  pl.BlockSpec((ep, 1), lambda: (0, 0))],
        out_specs=pl.BlockSpec((NP, NP), lambda: (0, 0)),
        out_shape=jax.ShapeDtypeStruct((NP, NP), F32),
    )(srcp, dstp)


def _gata(hg, w3, al, ar, cnt):
    bn, din = hg.shape
    dout = w3.shape[2]
    return pl.pallas_call(
        _gata_k,
        grid=(bn // NP,),
        in_specs=[_rows_spec(din), _const_spec(w3.shape), _const_spec(al.shape),
                  _const_spec(ar.shape), _const_spec(cnt.shape)],
        out_specs=_rows_spec(dout),
        out_shape=jax.ShapeDtypeStruct((bn, dout), F32),
    )(hg, w3, al, ar, cnt)


def _gatb(hg, w3, al, ar, cnt, hprev, res):
    bn, din = hg.shape
    dout = w3.shape[2]
    return pl.pallas_call(
        _gatb_k,
        grid=(bn // NP,),
        in_specs=[_rows_spec(din), _const_spec(w3.shape), _const_spec(al.shape),
                  _const_spec(ar.shape), _const_spec(cnt.shape),
                  _rows_spec(dout), _rows_spec(dout)],
        out_specs=_rows_spec(dout),
        out_shape=jax.ShapeDtypeStruct((bn, dout), F32),
    )(hg, w3, al, ar, cnt, hprev, res)


def _head(hl, wsk, bsk, w1, b1, w2, b2):
    bn = hl.shape[0]
    return pl.pallas_call(
        _head_k,
        grid=(bn // NP,),
        in_specs=[_rows_spec(SKC), _const_spec(wsk.shape), _const_spec(bsk.shape),
                  _const_spec(w1.shape), _const_spec(b1.shape),
                  _const_spec(w2.shape), _const_spec(b2.shape)],
        out_specs=_rows_spec(ODIM),
        out_shape=jax.ShapeDtypeStruct((bn, ODIM), F32),
    )(hl, wsk, bsk, w1, b1, w2, b2)


def kernel(x, params, edge_index):
    p = params
    B, _, N, T = x.shape
    BN = B * NP

    xt = jnp.transpose(x, (0, 2, 1, 3))
    xt = jnp.pad(xt, ((0, 0), (0, NP - N), (0, 0), (0, 0)))
    x0 = xt[:, :, 0, :].reshape(BN, T)
    x1 = xt[:, :, 1, :].reshape(BN, T)

    eyeT = jnp.eye(T, dtype=F32)
    sW = p['start_W'][:, 0, 0, 0]
    cW = p['cat_W'][:, 0, 0, 0]
    ss = (eyeT[:, None, :] * sW[None, :, None]).reshape(T, C * T)
    sc = (eyeT[:, None, :] * cW[None, :, None]).reshape(T, C * T)
    bs = jnp.repeat(p['start_b'], T)[None, :]
    bc = jnp.repeat(p['cat_b'], T)[None, :]
    h = _stem(x0, x1, ss, sc, bs, bc, BN)

    E = edge_index.shape[1]
    ep = ((E + 7) // 8) * 8
    pad = jnp.full((ep - E,), 255, jnp.int32)
    srcp = jnp.concatenate([edge_index[0], pad])[:, None]
    dstp = jnp.concatenate([edge_index[1], pad])[:, None]
    cnt = _mask(srcp, dstp)

    tcur = T
    hlasts = []
    for i in range(len(DIL)):
        di = DIL[i]
        tout = tcur - di
        e0 = jnp.eye(tcur, tout, dtype=F32)
        e1 = jnp.eye(tcur, tout, k=-di, dtype=F32)
        wf0 = p['filt_W'][i][:, :, 0, 0]
        wf1 = p['filt_W'][i][:, :, 0, 1]
        wg0 = p['gate_W'][i][:, :, 0, 0]
        wg1 = p['gate_W'][i][:, :, 0, 1]
        wfb = (jnp.einsum('oc,st->csot', wf0, e0)
               + jnp.einsum('oc,st->csot', wf1, e1)).reshape(C * tcur, C * tout)
        wgb = (jnp.einsum('oc,st->csot', wg0, e0)
               + jnp.einsum('oc,st->csot', wg1, e1)).reshape(C * tcur, C * tout)
        bf = jnp.repeat(p['filt_b'][i], tout)[None, :]
        bg = jnp.repeat(p['gate_b'][i], tout)[None, :]
        hin = h
        h = _tcn(h, wfb, wgb, bf, bg)
        hlasts.append(h.reshape(BN, C, tout)[:, :, tout - 1])
        if i == len(DIL) - 1:
            break
        d = C * tout
        wa = p['g%da_fcW' % i].reshape(H, d, d).transpose(0, 2, 1)
        hg = _gata(h, wa, p['g%da_al' % i], p['g%da_ar' % i], cnt)
        wb = p['g%db_fcW' % i].reshape(H, d, d).transpose(0, 2, 1)
        res = hin.reshape(BN, C, tcur)[:, :, tcur - tout:].reshape(BN, d)
        h = _gatb(hg, wb, p['g%db_al' % i], p['g%db_ar' % i], cnt, h, res)
        tcur = tout

    hl = jnp.concatenate(hlasts, axis=1)
    wsk = jnp.concatenate([p['skip_W'][i][:, :, 0, 0].T for i in range(len(DIL))],
                          axis=0)
    bsk = jnp.sum(p['skip_b'], axis=0)[None, :]
    w1 = p['end1_W'][:, :, 0, 0].T
    b1 = p['end1_b'][None, :]
    w2 = p['end2_W'][:, :, 0, 0].T
    b2 = p['end2_b'][None, :]
    out2d = _head(hl, wsk, bsk, w1, b1, w2, b2)

    out = out2d.reshape(B, NP, ODIM)[:, :N, :].transpose(0, 2, 1)[:, :, :, None]
    return out


# speedup vs baseline: 39.6697x; 1.3604x over previous
import jax
import jax.numpy as jnp
from jax.experimental import pallas as pl

H = 8
C = 40
SKC = 320
ENDC = 640
ODIM = 12
DIL = [1, 2, 1, 2, 1, 2, 1, 2]
NP = 208
INVBN = 1.0 / (1.0 + 1e-5) ** 0.5
F32 = jnp.float32


def _stem_k(x0_ref, x1_ref, ss_ref, sc_ref, bs_ref, bc_ref, out_ref):
    a = jnp.dot(x0_ref[...], ss_ref[...], preferred_element_type=F32) + bs_ref[...]
    b = jnp.dot(x1_ref[...], sc_ref[...], preferred_element_type=F32) + bc_ref[...]
    out_ref[...] = a + jnp.where(b >= 0, b, 0.01 * b)


def _tcn_k(h_ref, wf_ref, wg_ref, bf_ref, bg_ref, out_ref):
    hv = h_ref[...]
    f = jnp.tanh(jnp.dot(hv, wf_ref[...], preferred_element_type=F32) + bf_ref[...])
    g = jax.nn.sigmoid(jnp.dot(hv, wg_ref[...], preferred_element_type=F32) + bg_ref[...])
    out_ref[...] = f * g


def _mask_k(src_ref, dst_ref, out_ref):
    s = src_ref[...]
    d = dst_ref[...]
    iota = jax.lax.broadcasted_iota(jnp.int32, (s.shape[0], NP), 1)
    sh = (iota == s).astype(F32)
    dh = (iota == d).astype(F32)
    out_ref[...] = jax.lax.dot_general(
        dh, sh, (((0,), (0,)), ((), ())), preferred_element_type=F32)


def _gat2(hg, w3_ref, va_ref, vr_ref, cnt, valid, dout):
    elr = jnp.dot(hg, va_ref[...], preferred_element_type=F32)
    err = jnp.dot(hg, vr_ref[...], preferred_element_type=F32)
    elt = elr.T
    acc = jnp.zeros((NP, dout), F32)
    for h in range(H):
        w = w3_ref[h]
        feat = jax.lax.dot_general(hg, w, (((1,), (1,)), ((), ())),
                                   preferred_element_type=F32)
        e = err[:, h:h + 1] + elt[h:h + 1, :]
        e = jnp.maximum(e, 0.2 * e)
        e = jnp.where(valid, e, -1e30)
        m = jnp.max(e, axis=1, keepdims=True)
        sx = cnt * jnp.exp(e - m)
        ss = jnp.sum(sx, axis=1, keepdims=True)
        alpha = sx / jnp.where(ss > 0, ss, 1.0)
        rst = jnp.dot(alpha, feat, preferred_element_type=F32)
        acc = acc + jnp.where(rst > 0, rst, jnp.exp(jnp.minimum(rst, 0.0)) - 1.0)
    return acc * (1.0 / H)


def _layer_k(h_ref, res_ref, wf_ref, wg_ref, bf_ref, bg_ref, gsel_ref,
             wa3_ref, vaa_ref, vra_ref, wb3_ref, vab_ref, vrb_ref, cnt_ref,
             out_ref, hlast_ref):
    hv = h_ref[...]
    f = jnp.tanh(jnp.dot(hv, wf_ref[...], preferred_element_type=F32) + bf_ref[...])
    g = jax.nn.sigmoid(jnp.dot(hv, wg_ref[...], preferred_element_type=F32) + bg_ref[...])
    hn = f * g
    hlast_ref[...] = jnp.dot(hn, gsel_ref[...], preferred_element_type=F32)
    cnt = cnt_ref[...]
    valid = cnt > 0
    d = out_ref.shape[1]
    hga = _gat2(hn, wa3_ref, vaa_ref, vra_ref, cnt, valid, d)
    hgb = _gat2(hga, wb3_ref, vab_ref, vrb_ref, cnt, valid, d)
    out_ref[...] = (hgb + hn + res_ref[...]) * INVBN


def _head_k(hl_ref, wsk_ref, bsk_ref, w1_ref, b1_ref, w2_ref, b2_ref, out_ref):
    skip = jnp.dot(hl_ref[...], wsk_ref[...], preferred_element_type=F32) + bsk_ref[...]
    o = jnp.maximum(skip, 0.0)
    o = jnp.maximum(jnp.dot(o, w1_ref[...], preferred_element_type=F32) + b1_ref[...], 0.0)
    out_ref[...] = jnp.dot(o, w2_ref[...], preferred_element_type=F32) + b2_ref[...]


def _rows_spec(cols):
    return pl.BlockSpec((NP, cols), lambda r: (r, 0))


def _const_spec(shape):
    nd = len(shape)
    return pl.BlockSpec(shape, lambda r: (0,) * nd)


def _stem(x0, x1, ss, sc, bs, bc, bn):
    t = x0.shape[1]
    ct = ss.shape[1]
    return pl.pallas_call(
        _stem_k,
        grid=(bn // NP,),
        in_specs=[_rows_spec(t), _rows_spec(t), _const_spec(ss.shape),
                  _const_spec(sc.shape), _const_spec(bs.shape), _const_spec(bc.shape)],
        out_specs=_rows_spec(ct),
        out_shape=jax.ShapeDtypeStruct((bn, ct), F32),
    )(x0, x1, ss, sc, bs, bc)


def _tcn(h, wf, wg, bf, bg):
    bn, ctin = h.shape
    ctout = wf.shape[1]
    return pl.pallas_call(
        _tcn_k,
        grid=(bn // NP,),
        in_specs=[_rows_spec(ctin), _const_spec(wf.shape), _const_spec(wg.shape),
                  _const_spec(bf.shape), _const_spec(bg.shape)],
        out_specs=_rows_spec(ctout),
        out_shape=jax.ShapeDtypeStruct((bn, ctout), F32),
    )(h, wf, wg, bf, bg)


def _mask(srcp, dstp):
    ep = srcp.shape[0]
    return pl.pallas_call(
        _mask_k,
        in_specs=[pl.BlockSpec((ep, 1), lambda: (0, 0)),
                  pl.BlockSpec((ep, 1), lambda: (0, 0))],
        out_specs=pl.BlockSpec((NP, NP), lambda: (0, 0)),
        out_shape=jax.ShapeDtypeStruct((NP, NP), F32),
    )(srcp, dstp)


def _layer(h, res, wf, wg, bf, bg, gsel, wa3, vaa, vra, wb3, vab, vrb, cnt):
    bn, ctin = h.shape
    d = wf.shape[1]
    return pl.pallas_call(
        _layer_k,
        grid=(bn // NP,),
        in_specs=[_rows_spec(ctin), _rows_spec(d), _const_spec(wf.shape),
                  _const_spec(wg.shape), _const_spec(bf.shape), _const_spec(bg.shape),
                  _const_spec(gsel.shape), _const_spec(wa3.shape),
                  _const_spec(vaa.shape), _const_spec(vra.shape),
                  _const_spec(wb3.shape), _const_spec(vab.shape),
                  _const_spec(vrb.shape), _const_spec(cnt.shape)],
        out_specs=[_rows_spec(d), _rows_spec(C)],
        out_shape=[jax.ShapeDtypeStruct((bn, d), F32),
                   jax.ShapeDtypeStruct((bn, C), F32)],
    )(h, res, wf, wg, bf, bg, gsel, wa3, vaa, vra, wb3, vab, vrb, cnt)


def _head(hl, wsk, bsk, w1, b1, w2, b2):
    bn = hl.shape[0]
    return pl.pallas_call(
        _head_k,
        grid=(bn // NP,),
        in_specs=[_rows_spec(SKC), _const_spec(wsk.shape), _const_spec(bsk.shape),
                  _const_spec(w1.shape), _const_spec(b1.shape),
                  _const_spec(w2.shape), _const_spec(b2.shape)],
        out_specs=_rows_spec(ODIM),
        out_shape=jax.ShapeDtypeStruct((bn, ODIM), F32),
    )(hl, wsk, bsk, w1, b1, w2, b2)


def _conv_mats(p, i, tcur, tout, di):
    e0 = jnp.eye(tcur, tout, dtype=F32)
    e1 = jnp.eye(tcur, tout, k=-di, dtype=F32)
    wf0 = p['filt_W'][i][:, :, 0, 0]
    wf1 = p['filt_W'][i][:, :, 0, 1]
    wg0 = p['gate_W'][i][:, :, 0, 0]
    wg1 = p['gate_W'][i][:, :, 0, 1]
    wfb = (jnp.einsum('oc,st->csot', wf0, e0)
           + jnp.einsum('oc,st->csot', wf1, e1)).reshape(C * tcur, C * tout)
    wgb = (jnp.einsum('oc,st->csot', wg0, e0)
           + jnp.einsum('oc,st->csot', wg1, e1)).reshape(C * tcur, C * tout)
    bf = jnp.repeat(p['filt_b'][i], tout)[None, :]
    bg = jnp.repeat(p['gate_b'][i], tout)[None, :]
    return wfb, wgb, bf, bg


def kernel(x, params, edge_index):
    p = params
    B, _, N, T = x.shape
    BN = B * NP

    xt = jnp.transpose(x, (0, 2, 1, 3))
    xt = jnp.pad(xt, ((0, 0), (0, NP - N), (0, 0), (0, 0)))
    x0 = xt[:, :, 0, :].reshape(BN, T)
    x1 = xt[:, :, 1, :].reshape(BN, T)

    eyeT = jnp.eye(T, dtype=F32)
    sW = p['start_W'][:, 0, 0, 0]
    cW = p['cat_W'][:, 0, 0, 0]
    ss = (eyeT[:, None, :] * sW[None, :, None]).reshape(T, C * T)
    sc = (eyeT[:, None, :] * cW[None, :, None]).reshape(T, C * T)
    bs = jnp.repeat(p['start_b'], T)[None, :]
    bc = jnp.repeat(p['cat_b'], T)[None, :]
    h = _stem(x0, x1, ss, sc, bs, bc, BN)

    E = edge_index.shape[1]
    ep = ((E + 7) // 8) * 8
    pad = jnp.full((ep - E,), 255, jnp.int32)
    srcp = jnp.concatenate([edge_index[0], pad])[:, None]
    dstp = jnp.concatenate([edge_index[1], pad])[:, None]
    cnt = _mask(srcp, dstp)

    tcur = T
    hlasts = []
    for i in range(len(DIL)):
        di = DIL[i]
        tout = tcur - di
        wfb, wgb, bf, bg = _conv_mats(p, i, tcur, tout, di)
        if i == len(DIL) - 1:
            h = _tcn(h, wfb, wgb, bf, bg)
            hlasts.append(h.reshape(BN, C, tout)[:, :, tout - 1])
            break
        d = C * tout
        gsel = jnp.zeros((C, tout, C), F32).at[:, tout - 1, :].set(jnp.eye(C, dtype=F32))
        gsel = gsel.reshape(C * tout, C)
        wa3 = p['g%da_fcW' % i].reshape(H, d, d)
        wb3 = p['g%db_fcW' % i].reshape(H, d, d)
        vaa = jnp.einsum('hoi,ho->ih', wa3, p['g%da_al' % i])
        vra = jnp.einsum('hoi,ho->ih', wa3, p['g%da_ar' % i])
        vab = jnp.einsum('hoi,ho->ih', wb3, p['g%db_al' % i])
        vrb = jnp.einsum('hoi,ho->ih', wb3, p['g%db_ar' % i])
        res = h.reshape(BN, C, tcur)[:, :, tcur - tout:].reshape(BN, d)
        h, hlast = _layer(h, res, wfb, wgb, bf, bg, gsel,
                          wa3, vaa, vra, wb3, vab, vrb, cnt)
        hlasts.append(hlast)
        tcur = tout

    hl = jnp.concatenate(hlasts, axis=1)
    wsk = jnp.concatenate([p['skip_W'][i][:, :, 0, 0].T for i in range(len(DIL))],
                          axis=0)
    bsk = jnp.sum(p['skip_b'], axis=0)[None, :]
    w1 = p['end1_W'][:, :, 0, 0].T
    b1 = p['end1_b'][None, :]
    w2 = p['end2_W'][:, :, 0, 0].T
    b2 = p['end2_b'][None, :]
    out2d = _head(hl, wsk, bsk, w1, b1, w2, b2)

    out = out2d.reshape(B, NP, ODIM)[:, :N, :].transpose(0, 2, 1)[:, :, :, None]
    return out


# speedup vs baseline: 45.2572x; 1.1409x over previous
import jax
import jax.numpy as jnp
from jax.experimental import pallas as pl

H = 8
C = 40
SKC = 320
ENDC = 640
ODIM = 12
DIL = [1, 2, 1, 2, 1, 2, 1, 2]
NP = 208
INVBN = 1.0 / (1.0 + 1e-5) ** 0.5
F32 = jnp.float32


def _stem_k(x0_ref, x1_ref, ss_ref, sc_ref, bs_ref, bc_ref, out_ref):
    a = jnp.dot(x0_ref[...], ss_ref[...], preferred_element_type=F32) + bs_ref[...]
    b = jnp.dot(x1_ref[...], sc_ref[...], preferred_element_type=F32) + bc_ref[...]
    out_ref[...] = a + jnp.where(b >= 0, b, 0.01 * b)


def _tcn_k(h_ref, wf_ref, wg_ref, bf_ref, bg_ref, out_ref):
    hv = h_ref[...]
    f = jnp.tanh(jnp.dot(hv, wf_ref[...], preferred_element_type=F32) + bf_ref[...])
    g = jax.nn.sigmoid(jnp.dot(hv, wg_ref[...], preferred_element_type=F32) + bg_ref[...])
    out_ref[...] = f * g


def _mask_k(src_ref, dst_ref, out_ref):
    s = src_ref[...]
    d = dst_ref[...]
    iota = jax.lax.broadcasted_iota(jnp.int32, (s.shape[0], NP), 1)
    sh = (iota == s).astype(F32)
    dh = (iota == d).astype(F32)
    out_ref[...] = jax.lax.dot_general(
        dh, sh, (((0,), (0,)), ((), ())), preferred_element_type=F32)


def _gat2(hg, w3_ref, vv_ref, cnt, valid, dout, nb):
    ee = jnp.dot(hg, vv_ref[...], preferred_element_type=F32)
    parts = []
    for b in range(nb):
        eb = ee[b * NP:(b + 1) * NP, :]
        parts.append((eb[:, :H].T, eb[:, H:]))
    accs = [jnp.zeros((NP, dout), F32) for _ in range(nb)]
    for h in range(H):
        w = w3_ref[h]
        feat = jax.lax.dot_general(hg, w, (((1,), (1,)), ((), ())),
                                   preferred_element_type=F32)
        for b in range(nb):
            elt, err = parts[b]
            e = err[:, h:h + 1] + elt[h:h + 1, :]
            e = jnp.maximum(e, 0.2 * e)
            e = jnp.where(valid, e, -1e30)
            m = jnp.max(e, axis=1, keepdims=True)
            sx = cnt * jnp.exp(e - m)
            ss = jnp.sum(sx, axis=1, keepdims=True)
            alpha = sx / jnp.where(ss > 0, ss, 1.0)
            rst = jnp.dot(alpha, feat[b * NP:(b + 1) * NP, :],
                          preferred_element_type=F32)
            accs[b] = accs[b] + jnp.where(rst > 0, rst,
                                          jnp.exp(jnp.minimum(rst, 0.0)) - 1.0)
    return jnp.concatenate(accs, axis=0) * (1.0 / H)


def _layer_k(h_ref, res_ref, wf_ref, wg_ref, bf_ref, bg_ref, gsel_ref,
             wa3_ref, vva_ref, wb3_ref, vvb_ref, cnt_ref,
             out_ref, hlast_ref):
    hv = h_ref[...]
    f = jnp.tanh(jnp.dot(hv, wf_ref[...], preferred_element_type=F32) + bf_ref[...])
    g = jax.nn.sigmoid(jnp.dot(hv, wg_ref[...], preferred_element_type=F32) + bg_ref[...])
    hn = f * g
    hlast_ref[...] = jnp.dot(hn, gsel_ref[...], preferred_element_type=F32)
    cnt = cnt_ref[...]
    valid = cnt > 0
    d = out_ref.shape[1]
    nb = out_ref.shape[0] // NP
    hga = _gat2(hn, wa3_ref, vva_ref, cnt, valid, d, nb)
    hgb = _gat2(hga, wb3_ref, vvb_ref, cnt, valid, d, nb)
    out_ref[...] = (hgb + hn + res_ref[...]) * INVBN


def _head_k(hl_ref, wsk_ref, bsk_ref, w1_ref, b1_ref, w2_ref, b2_ref, out_ref):
    skip = jnp.dot(hl_ref[...], wsk_ref[...], preferred_element_type=F32) + bsk_ref[...]
    o = jnp.maximum(skip, 0.0)
    o = jnp.maximum(jnp.dot(o, w1_ref[...], preferred_element_type=F32) + b1_ref[...], 0.0)
    out_ref[...] = jnp.dot(o, w2_ref[...], preferred_element_type=F32) + b2_ref[...]


def _call(body, outs, *args):
    return pl.pallas_call(
        body,
        in_specs=[pl.BlockSpec(a.shape, lambda *_, _n=a.ndim: (0,) * _n)
                  for a in args],
        out_specs=jax.tree.map(
            lambda s: pl.BlockSpec(s.shape, lambda *_: (0,) * len(s.shape)), outs),
        out_shape=outs,
    )(*args)


def _stem(x0, x1, ss, sc, bs, bc, bn):
    ct = ss.shape[1]
    return _call(_stem_k, jax.ShapeDtypeStruct((bn, ct), F32),
                 x0, x1, ss, sc, bs, bc)


def _tcn(h, wf, wg, bf, bg):
    bn = h.shape[0]
    ctout = wf.shape[1]
    return _call(_tcn_k, jax.ShapeDtypeStruct((bn, ctout), F32),
                 h, wf, wg, bf, bg)


def _mask(srcp, dstp):
    return _call(_mask_k, jax.ShapeDtypeStruct((NP, NP), F32), srcp, dstp)


def _layer(h, res, wf, wg, bf, bg, gsel, wa3, vva, wb3, vvb, cnt):
    bn = h.shape[0]
    d = wf.shape[1]
    return _call(_layer_k,
                 [jax.ShapeDtypeStruct((bn, d), F32),
                  jax.ShapeDtypeStruct((bn, C), F32)],
                 h, res, wf, wg, bf, bg, gsel, wa3, vva, wb3, vvb, cnt)


def _head(hl, wsk, bsk, w1, b1, w2, b2):
    bn = hl.shape[0]
    return _call(_head_k, jax.ShapeDtypeStruct((bn, ODIM), F32),
                 hl, wsk, bsk, w1, b1, w2, b2)


def _conv_mats(p, i, tcur, tout, di):
    e0 = jnp.eye(tcur, tout, dtype=F32)
    e1 = jnp.eye(tcur, tout, k=-di, dtype=F32)
    wf0 = p['filt_W'][i][:, :, 0, 0]
    wf1 = p['filt_W'][i][:, :, 0, 1]
    wg0 = p['gate_W'][i][:, :, 0, 0]
    wg1 = p['gate_W'][i][:, :, 0, 1]
    wfb = (jnp.einsum('oc,st->csot', wf0, e0)
           + jnp.einsum('oc,st->csot', wf1, e1)).reshape(C * tcur, C * tout)
    wgb = (jnp.einsum('oc,st->csot', wg0, e0)
           + jnp.einsum('oc,st->csot', wg1, e1)).reshape(C * tcur, C * tout)
    bf = jnp.repeat(p['filt_b'][i], tout)[None, :]
    bg = jnp.repeat(p['gate_b'][i], tout)[None, :]
    return wfb, wgb, bf, bg


def kernel(x, params, edge_index):
    p = params
    B, _, N, T = x.shape
    BN = B * NP

    xt = jnp.transpose(x, (0, 2, 1, 3))
    xt = jnp.pad(xt, ((0, 0), (0, NP - N), (0, 0), (0, 0)))
    x0 = xt[:, :, 0, :].reshape(BN, T)
    x1 = xt[:, :, 1, :].reshape(BN, T)

    eyeT = jnp.eye(T, dtype=F32)
    sW = p['start_W'][:, 0, 0, 0]
    cW = p['cat_W'][:, 0, 0, 0]
    ss = (eyeT[:, None, :] * sW[None, :, None]).reshape(T, C * T)
    sc = (eyeT[:, None, :] * cW[None, :, None]).reshape(T, C * T)
    bs = jnp.repeat(p['start_b'], T)[None, :]
    bc = jnp.repeat(p['cat_b'], T)[None, :]
    h = _stem(x0, x1, ss, sc, bs, bc, BN)

    E = edge_index.shape[1]
    ep = ((E + 7) // 8) * 8
    pad = jnp.full((ep - E,), 255, jnp.int32)
    srcp = jnp.concatenate([edge_index[0], pad])[:, None]
    dstp = jnp.concatenate([edge_index[1], pad])[:, None]
    cnt = _mask(srcp, dstp)

    tcur = T
    hlasts = []
    for i in range(len(DIL)):
        di = DIL[i]
        tout = tcur - di
        wfb, wgb, bf, bg = _conv_mats(p, i, tcur, tout, di)
        if i == len(DIL) - 1:
            h = _tcn(h, wfb, wgb, bf, bg)
            hlasts.append(h.reshape(BN, C, tout)[:, :, tout - 1])
            break
        d = C * tout
        gsel = jnp.zeros((C, tout, C), F32).at[:, tout - 1, :].set(jnp.eye(C, dtype=F32))
        gsel = gsel.reshape(C * tout, C)
        wa3 = p['g%da_fcW' % i].reshape(H, d, d)
        wb3 = p['g%db_fcW' % i].reshape(H, d, d)
        ala = jnp.stack([p['g%da_al' % i], p['g%da_ar' % i]], axis=1)
        alb = jnp.stack([p['g%db_al' % i], p['g%db_ar' % i]], axis=1)
        vva = jnp.einsum('hoi,hso->ish', wa3, ala).reshape(d, 2 * H)
        vvb = jnp.einsum('hoi,hso->ish', wb3, alb).reshape(d, 2 * H)
        res = h.reshape(BN, C, tcur)[:, :, tcur - tout:].reshape(BN, d)
        h, hlast = _layer(h, res, wfb, wgb, bf, bg, gsel,
                          wa3, vva, wb3, vvb, cnt)
        hlasts.append(hlast)
        tcur = tout

    hl = jnp.concatenate(hlasts, axis=1)
    wsk = jnp.concatenate([p['skip_W'][i][:, :, 0, 0].T for i in range(len(DIL))],
                          axis=0)
    bsk = jnp.sum(p['skip_b'], axis=0)[None, :]
    w1 = p['end1_W'][:, :, 0, 0].T
    b1 = p['end1_b'][None, :]
    w2 = p['end2_W'][:, :, 0, 0].T
    b2 = p['end2_b'][None, :]
    out2d = _head(hl, wsk, bsk, w1, b1, w2, b2)

    out = out2d.reshape(B, NP, ODIM)[:, :N, :].transpose(0, 2, 1)[:, :, :, None]
    return out


# speedup vs baseline: 50.7151x; 1.1206x over previous
import functools

import numpy as np
import jax
import jax.numpy as jnp
from jax.experimental import pallas as pl

H = 8
C = 40
SKC = 320
ENDC = 640
ODIM = 12
DIL = [1, 2, 1, 2, 1, 2, 1, 2]
NP = 208
NB = 8
INVBN = 1.0 / (1.0 + 1e-5) ** 0.5
F32 = jnp.float32
_DG = jax.lax.dot_general


def _dgt(a, b):
    return _DG(a, b, (((1,), (1,)), ((), ())), preferred_element_type=F32)


def _ii(shape, dim):
    return jax.lax.broadcasted_iota(jnp.int32, shape, dim)


def _stem_k(x0_ref, x1_ref, sw_ref, cw_ref, bs_ref, bc_ref, out_ref, *, t):
    mask = (_ii((t, C * t), 0) == _ii((t, C * t), 1) % t).astype(F32)
    rcs = (_ii((C * t, C), 0) // t == _ii((C * t, C), 1)).astype(F32)
    ssw = mask * _dgt(sw_ref[...], rcs)
    scw = mask * _dgt(cw_ref[...], rcs)
    a = jnp.dot(x0_ref[...], ssw, preferred_element_type=F32) + _dgt(bs_ref[...], rcs)
    b = jnp.dot(x1_ref[...], scw, preferred_element_type=F32) + _dgt(bc_ref[...], rcs)
    out_ref[...] = a + jnp.where(b >= 0, b, 0.01 * b)


def _mask_k(src_ref, dst_ref, out_ref):
    s = src_ref[...]
    d = dst_ref[...]
    iota = jax.lax.broadcasted_iota(jnp.int32, (s.shape[0], NP), 1)
    sh = (iota == s).astype(F32)
    dh = (iota == d).astype(F32)
    c = _DG(dh, sh, (((0,), (0,)), ((), ())),
            preferred_element_type=F32)
    out_ref[...] = jnp.where(c > 0.5, jnp.log(c), -1e30)


def _gat2(hg, w2d_ref, ala_ref, ara_ref, lcnt, dout):
    w2d = w2d_ref[...]
    hd = H * dout
    blk2 = (_ii((2 * H, hd), 1) // dout ==
            _ii((2 * H, hd), 0) % H).astype(F32)
    alar = jnp.concatenate([ala_ref[...], ara_ref[...]], axis=0)
    m2 = jnp.tile(alar, (1, H)) * blk2
    vlr = jnp.dot(m2, w2d, preferred_element_type=F32)
    elT = _dgt(vlr[:H], hg)
    err = _dgt(hg, vlr[H:])
    accs = [jnp.zeros((NP, dout), F32) for _ in range(NB)]
    for h in range(H):
        feat = _dgt(hg, w2d[h * dout:(h + 1) * dout, :])
        for b in range(NB):
            elrow = elT[h:h + 1, b * NP:(b + 1) * NP]
            ercol = err[b * NP:(b + 1) * NP, h:h + 1]
            zm = ercol + jnp.max(elrow)
            m = jnp.maximum(zm, 0.2 * zm)
            e = ercol + elrow
            e = jnp.maximum(e, 0.2 * e)
            sx = jnp.exp(e - m + lcnt)
            ss = jnp.sum(sx, axis=1, keepdims=True)
            rs = 1.0 / jnp.where(ss > 0, ss, 1.0)
            num = jnp.dot(sx, feat[b * NP:(b + 1) * NP, :],
                          preferred_element_type=F32)
            rst = num * rs
            accs[b] = accs[b] + (jnp.maximum(rst, 0.0) +
                                 jnp.exp(jnp.minimum(rst, 0.0)) - 1.0)
    return jnp.concatenate(accs, axis=0) * (1.0 / H)


def _layer_body(h_ref, res_ref, wf_ref, wg_ref, bf_ref, bg_ref,
                wa_ref, ala_ref, ara_ref, wb_ref, alb_ref, arb_ref, lcnt_ref,
                out_ref, hlast_ref, *, tcur, tout, di, last):
    rr = (_ii((C * tcur, C), 0) // tcur == _ii((C * tcur, C), 1)).astype(F32)
    rc = (_ii((C * tout, C), 0) // tout == _ii((C * tout, C), 1)).astype(F32)
    sidx = _ii((C * tcur, C * tout), 0) % tcur
    tidx = _ii((C * tcur, C * tout), 1) % tout
    m0 = (sidx == tidx).astype(F32)
    m1 = (sidx == tidx + di).astype(F32)
    hv = h_ref[...]
    wf0, wf1 = wf_ref[0], wf_ref[1]
    wg0, wg1 = wg_ref[0], wg_ref[1]
    wfb = _dgt(_dgt(rr, wf0), rc) * m0 + _dgt(_dgt(rr, wf1), rc) * m1
    wgb = _dgt(_dgt(rr, wg0), rc) * m0 + _dgt(_dgt(rr, wg1), rc) * m1
    bft = _dgt(bf_ref[...], rc)
    bgt = _dgt(bg_ref[...], rc)
    f = jnp.tanh(jnp.dot(hv, wfb, preferred_element_type=F32) + bft)
    g = jax.nn.sigmoid(jnp.dot(hv, wgb, preferred_element_type=F32) + bgt)
    hn = f * g
    gsel = rc * (_ii((C * tout, C), 0) % tout == tout - 1).astype(F32)
    hlast_ref[...] = jnp.dot(hn, gsel, preferred_element_type=F32)
    if last:
        out_ref[...] = hn
        return
    lcnt = lcnt_ref[...]
    d = C * tout
    hga = _gat2(hn, wa_ref, ala_ref, ara_ref, lcnt, d)
    hgb = _gat2(hga, wb_ref, alb_ref, arb_ref, lcnt, d)
    out_ref[...] = (hgb + hn + res_ref[...]) * INVBN


def _head_k(hl_ref, wsk_ref, bsk_ref, w1_ref, b1_ref, w2_ref, b2_ref, out_ref):
    skip = _dgt(hl_ref[...], wsk_ref[...]) + bsk_ref[...]
    o = jnp.maximum(skip, 0.0)
    o = jnp.maximum(_dgt(o, w1_ref[...]) + b1_ref[...], 0.0)
    out_ref[...] = _dgt(o, w2_ref[...]) + b2_ref[...]


def _call(body, outs, *args):
    return pl.pallas_call(
        body,
        in_specs=[pl.BlockSpec(a.shape, lambda *_, _n=a.ndim: (0,) * _n)
                  for a in args],
        out_specs=jax.tree.map(
            lambda s: pl.BlockSpec(s.shape, lambda *_: (0,) * len(s.shape)), outs),
        out_shape=outs,
    )(*args)


def kernel(x, params, edge_index):
    p = params
    B, _, N, T = x.shape
    BN = B * NP

    xt = jnp.transpose(x, (0, 2, 1, 3))
    xt = jnp.pad(xt, ((0, 0), (0, NP - N), (0, 0), (0, 0)))
    x0 = xt[:, :, 0, :].reshape(BN, T)
    x1 = xt[:, :, 1, :].reshape(BN, T)

    h = _call(functools.partial(_stem_k, t=T),
              jax.ShapeDtypeStruct((BN, C * T), F32),
              x0, x1, p['start_W'].reshape(1, C), p['cat_W'].reshape(1, C),
              p['start_b'][None, :], p['cat_b'][None, :])

    E = edge_index.shape[1]
    ep = ((E + 7) // 8) * 8
    pad = jnp.full((ep - E,), 255, jnp.int32)
    srcp = jnp.concatenate([edge_index[0], pad])[:, None]
    dstp = jnp.concatenate([edge_index[1], pad])[:, None]
    lcnt = _call(_mask_k, jax.ShapeDtypeStruct((NP, NP), F32), srcp, dstp)

    tcur = T
    hlasts = []
    for i in range(len(DIL)):
        di = DIL[i]
        tout = tcur - di
        d = C * tout
        last = i == len(DIL) - 1
        wf2 = p['filt_W'][i][:, :, 0, :].transpose(2, 0, 1)
        wg2 = p['gate_W'][i][:, :, 0, :].transpose(2, 0, 1)
        body = functools.partial(_layer_body, tcur=tcur, tout=tout, di=di,
                                 last=last)
        outs = [jax.ShapeDtypeStruct((BN, d), F32),
                jax.ShapeDtypeStruct((BN, C), F32)]
        if last:
            z = jnp.zeros((1, 1), F32)
            h, hlast = _call(body, outs, h, z, wf2, wg2,
                             p['filt_b'][i][None, :], p['gate_b'][i][None, :],
                             z, z, z, z, z, z, z)
        else:
            res = h.reshape(BN, C, tcur)[:, :, tcur - tout:].reshape(BN, d)
            h, hlast = _call(
                body, outs, h, res, wf2, wg2,
                p['filt_b'][i][None, :], p['gate_b'][i][None, :],
                p['g%da_fcW' % i], p['g%da_al' % i], p['g%da_ar' % i],
                p['g%db_fcW' % i], p['g%db_al' % i], p['g%db_ar' % i], lcnt)
        hlasts.append(hlast)
        tcur = tout

    hl = jnp.concatenate(hlasts, axis=1)
    wskc = jnp.concatenate([p['skip_W'][i][:, :, 0, 0] for i in range(len(DIL))],
                           axis=1)
    bsk = jnp.sum(p['skip_b'], axis=0)[None, :]
    out2d = _call(_head_k, jax.ShapeDtypeStruct((BN, ODIM), F32),
                  hl, wskc, bsk, p['end1_W'][:, :, 0, 0], p['end1_b'][None, :],
                  p['end2_W'][:, :, 0, 0], p['end2_b'][None, :])

    out = out2d.reshape(B, NP, ODIM)[:, :N, :].transpose(0, 2, 1)[:, :, :, None]
    return out
